# baseline (speedup 1.0000x reference)
import functools

import jax
import jax.numpy as jnp
from jax import lax
from jax.experimental import pallas as pl
from jax.experimental.pallas import tpu as pltpu
from jax.experimental.pallas import tpu_sc as plsc

_NC = 2
_NS = 16
_NW = _NC * _NS
_CHUNK = 128
_R = 256


def _sc_mesh():
    return plsc.VectorSubcoreMesh(core_axis_name="c", subcore_axis_name="s")


def _make_deg_kernel(NP, CPW):
    rows_pt = NP // _NS
    zc = rows_pt // _CHUNK

    @functools.partial(
        pl.kernel,
        out_type=jax.ShapeDtypeStruct((_NC, NP), jnp.float32),
        mesh=_sc_mesh(),
        scratch_types=[
            pltpu.VMEM((CPW, _CHUNK), jnp.int32),
            pltpu.VMEM((_CHUNK,), jnp.float32),
            pltpu.VMEM((_CHUNK,), jnp.float32),
            pltpu.VMEM((rows_pt,), jnp.float32),
            pltpu.VMEM_SHARED((NP,), jnp.float32),
        ],
    )
    def deg_kernel(dst_hbm, out_hbm, didx, ones_v, zeros_v, stage_v, acc_sp):
        c = lax.axis_index("c")
        s = lax.axis_index("s")
        w = c * _NS + s
        for j in range(_CHUNK // 16):
            ones_v[pl.ds(j * 16, 16)] = jnp.full((16,), 1.0, jnp.float32)
            zeros_v[pl.ds(j * 16, 16)] = jnp.zeros((16,), jnp.float32)
        base = pl.multiple_of(s * rows_pt, _CHUNK)
        for t in range(zc):
            pltpu.sync_copy(zeros_v, acc_sp.at[pl.ds(base + t * _CHUNK, _CHUNK)])
        plsc.subcore_barrier()
        pltpu.sync_copy(dst_hbm.at[pl.ds(w * CPW, CPW)], didx)

        def body(ci, carry):
            pltpu.sync_copy(ones_v, acc_sp.at[didx.at[ci]], add=True)
            return carry

        lax.fori_loop(0, CPW, body, 0)
        plsc.subcore_barrier()
        pltpu.sync_copy(acc_sp.at[pl.ds(base, rows_pt)], stage_v)
        pltpu.sync_copy(stage_v, out_hbm.at[c, pl.ds(base, rows_pt)])

    return deg_kernel


def _make_agg_kernel(NP, H, CPW):
    rows_pt = NP // _NS
    zc = rows_pt // _CHUNK

    NB = 4

    @functools.partial(
        pl.kernel,
        out_type=jax.ShapeDtypeStruct((_NC, NP, H), jnp.float32),
        mesh=_sc_mesh(),
        compiler_params=pltpu.CompilerParams(use_tc_tiling_on_sc=False),
        scratch_types=[
            pltpu.VMEM((CPW, _CHUNK), jnp.int32),
            pltpu.VMEM((CPW, _CHUNK), jnp.int32),
            [pltpu.VMEM((_CHUNK, H), jnp.float32) for _ in range(NB)],
            pltpu.VMEM_SHARED((NP, H), jnp.float32),
            [pltpu.SemaphoreType.DMA for _ in range(NB)],
            [pltpu.SemaphoreType.DMA for _ in range(NB)],
        ],
    )
    def agg_kernel(u_hbm, src_hbm, dst_hbm, out_hbm,
                   sidx, didx, bufs, acc_sp, gsems, tsems):
        c = lax.axis_index("c")
        s = lax.axis_index("s")
        w = c * _NS + s

        def zrow(i, carry):
            for j in range(H // 16):
                bufs[0][i, pl.ds(j * 16, 16)] = jnp.zeros((16,), jnp.float32)
            return carry

        lax.fori_loop(0, _CHUNK, zrow, 0)
        base = pl.multiple_of(s * rows_pt, _CHUNK)
        for t in range(zc):
            pltpu.sync_copy(bufs[0],
                            acc_sp.at[pl.ds(base + t * _CHUNK, _CHUNK)])
        plsc.subcore_barrier()
        pltpu.sync_copy(src_hbm.at[pl.ds(w * CPW, CPW)], sidx)
        pltpu.sync_copy(dst_hbm.at[pl.ds(w * CPW, CPW)], didx)

        rounds = CPW // NB

        def gathers(p):
            for j in range(NB):
                pltpu.async_copy(u_hbm.at[sidx.at[p * NB + j]], bufs[j],
                                 gsems[j])

        def scatters(p):
            for j in range(NB):
                pltpu.make_async_copy(u_hbm.at[sidx.at[0]], bufs[j],
                                      gsems[j]).wait()
                pltpu.async_copy(bufs[j], acc_sp.at[didx.at[p * NB + j]],
                                 tsems[j], add=True)

        gathers(0)

        def body(p, carry):
            scatters(p)
            for j in range(NB):
                pltpu.make_async_copy(bufs[j], acc_sp.at[didx.at[0]],
                                      tsems[j]).wait()
                pltpu.async_copy(u_hbm.at[sidx.at[(p + 1) * NB + j]],
                                 bufs[j], gsems[j])
            return carry

        lax.fori_loop(0, rounds - 1, body, 0)
        scatters(rounds - 1)
        for j in range(NB):
            pltpu.make_async_copy(bufs[j], acc_sp.at[didx.at[0]],
                                  tsems[j]).wait()
        plsc.subcore_barrier()

        def copyout(t, carry):
            off = pl.multiple_of(base + t * _CHUNK, _CHUNK)
            pltpu.sync_copy(acc_sp.at[pl.ds(off, _CHUNK)], bufs[0])
            pltpu.sync_copy(bufs[0], out_hbm.at[c, pl.ds(off, _CHUNK)])
            return carry

        lax.fori_loop(0, zc, copyout, 0)

    return agg_kernel


def _tc_pre(xp, W1, deg0, deg1, NP, N):
    D = xp.shape[1]
    H = W1.shape[1]

    def body(x_ref, w_ref, d0_ref, d1_ref, dinv_ref, u_ref):
        i = pl.program_id(0)
        d = d0_ref[...] + d1_ref[...] + 1.0
        rows = lax.broadcasted_iota(jnp.int32, (_R, 1), 0) + i * _R
        dinv = jnp.where(rows < N, lax.rsqrt(d), 0.0)
        dinv_ref[...] = dinv
        u_ref[...] = dinv * jnp.dot(x_ref[...], w_ref[...],
                                    preferred_element_type=jnp.float32)

    return pl.pallas_call(
        body,
        grid=(NP // _R,),
        in_specs=[
            pl.BlockSpec((_R, D), lambda i: (i, 0)),
            pl.BlockSpec((D, H), lambda i: (0, 0)),
            pl.BlockSpec((_R, 1), lambda i: (i, 0)),
            pl.BlockSpec((_R, 1), lambda i: (i, 0)),
        ],
        out_specs=[
            pl.BlockSpec((_R, 1), lambda i: (i, 0)),
            pl.BlockSpec((_R, H), lambda i: (i, 0)),
        ],
        out_shape=[
            jax.ShapeDtypeStruct((NP, 1), jnp.float32),
            jax.ShapeDtypeStruct((NP, H), jnp.float32),
        ],
    )(xp, W1, deg0, deg1)


def _tc_layer(p, u, dinv, b, Wn, NP):
    H = u.shape[1]
    Hn = Wn.shape[1]

    def body(p0_ref, p1_ref, u_ref, dinv_ref, b_ref, w_ref, un_ref):
        dinv = dinv_ref[...]
        h = jnp.maximum((p0_ref[...] + p1_ref[...] + u_ref[...]) * dinv
                        + b_ref[...], 0.0)
        un_ref[...] = dinv * jnp.dot(h, w_ref[...],
                                     preferred_element_type=jnp.float32)

    return pl.pallas_call(
        body,
        grid=(NP // _R,),
        in_specs=[
            pl.BlockSpec((_R, H), lambda i: (i, 0)),
            pl.BlockSpec((_R, H), lambda i: (i, 0)),
            pl.BlockSpec((_R, H), lambda i: (i, 0)),
            pl.BlockSpec((_R, 1), lambda i: (i, 0)),
            pl.BlockSpec((1, H), lambda i: (0, 0)),
            pl.BlockSpec((H, Hn), lambda i: (0, 0)),
        ],
        out_specs=pl.BlockSpec((_R, Hn), lambda i: (i, 0)),
        out_shape=jax.ShapeDtypeStruct((NP, Hn), jnp.float32),
    )(p[0], p[1], u, dinv, b, Wn)


def _tc_head(p, u, dinv, b3, Wp1, bp1, Wp2, bp2, NP):
    H = u.shape[1]
    Hh = Wp1.shape[1]

    def body(p0_ref, p1_ref, u_ref, dinv_ref, b3_ref, w1_ref, b1_ref,
             w2_ref, b2_ref, out_ref):
        dinv = dinv_ref[...]
        h3 = jnp.maximum((p0_ref[...] + p1_ref[...] + u_ref[...]) * dinv
                         + b3_ref[...], 0.0)
        hp = jnp.maximum(jnp.dot(h3, w1_ref[...],
                                 preferred_element_type=jnp.float32)
                         + b1_ref[...], 0.0)
        out_ref[...] = jnp.dot(hp, w2_ref[...],
                               preferred_element_type=jnp.float32) + b2_ref[...]

    return pl.pallas_call(
        body,
        grid=(NP // _R,),
        in_specs=[
            pl.BlockSpec((_R, H), lambda i: (i, 0)),
            pl.BlockSpec((_R, H), lambda i: (i, 0)),
            pl.BlockSpec((_R, H), lambda i: (i, 0)),
            pl.BlockSpec((_R, 1), lambda i: (i, 0)),
            pl.BlockSpec((1, H), lambda i: (0, 0)),
            pl.BlockSpec((H, Hh), lambda i: (0, 0)),
            pl.BlockSpec((1, Hh), lambda i: (0, 0)),
            pl.BlockSpec((Hh, 1), lambda i: (0, 0)),
            pl.BlockSpec((1, 1), lambda i: (0, 0)),
        ],
        out_specs=pl.BlockSpec((_R, 1), lambda i: (i, 0)),
        out_shape=jax.ShapeDtypeStruct((NP, 1), jnp.float32),
    )(p[0], p[1], u, dinv, b3, Wp1, bp1, Wp2, bp2)


def kernel(x, edge_index, batch, W1, b1, W2, b2, W3, b3, Wp1, bp1, Wp2, bp2):
    N, D = x.shape
    H = W1.shape[1]
    E = edge_index.shape[1]

    NP = ((N + 1 + _R - 1) // _R) * _R
    grain = _NW * _CHUNK * 8
    EP = ((E + grain - 1) // grain) * grain
    CPW = EP // (_NW * _CHUNK)

    xp = jnp.pad(x, ((0, NP - N), (0, 0)))
    pad = EP - E
    srcp = jnp.concatenate(
        [edge_index[0], jnp.full((pad,), N, jnp.int32)]).reshape(EP // _CHUNK,
                                                                 _CHUNK)
    dstp = jnp.concatenate(
        [edge_index[1], jnp.full((pad,), N, jnp.int32)]).reshape(EP // _CHUNK,
                                                                 _CHUNK)
    b1r = b1.reshape(1, H)
    b2r = b2.reshape(1, H)
    b3r = b3.reshape(1, H)
    bp1r = bp1.reshape(1, -1)
    bp2r = bp2.reshape(1, 1)

    deg_p = _make_deg_kernel(NP, CPW)(dstp)
    deg01 = deg_p.reshape(_NC, NP, 1)

    dinv, u1 = _tc_pre(xp, W1, deg01[0], deg01[1], NP, N)

    agg = _make_agg_kernel(NP, H, CPW)

    p1_ = agg(u1, srcp, dstp)
    u2 = _tc_layer(p1_, u1, dinv, b1r, W2, NP)
    p2_ = agg(u2, srcp, dstp)
    u3 = _tc_layer(p2_, u2, dinv, b2r, W3, NP)
    p3_ = agg(u3, srcp, dstp)
    out = _tc_head(p3_, u3, dinv, b3r, Wp1, bp1r, Wp2, bp2r, NP)
    return out[:N]


# speedup vs baseline: 2.8872x; 2.8872x over previous
import functools

import jax
import jax.numpy as jnp
from jax import lax
from jax.experimental import pallas as pl
from jax.experimental.pallas import tpu as pltpu
from jax.experimental.pallas import tpu_sc as plsc

_NC = 2
_NS = 16
_NW = _NC * _NS
_CHUNK = 128
_R = 256


def _sc_mesh():
    return plsc.VectorSubcoreMesh(core_axis_name="c", subcore_axis_name="s")


def _make_deg_kernel(NP, CPW):
    rows_pt = NP // _NS
    zc = rows_pt // _CHUNK

    @functools.partial(
        pl.kernel,
        out_type=jax.ShapeDtypeStruct((_NC, NP), jnp.float32),
        mesh=_sc_mesh(),
        scratch_types=[
            pltpu.VMEM((CPW, _CHUNK), jnp.int32),
            pltpu.VMEM((_CHUNK,), jnp.float32),
            pltpu.VMEM((_CHUNK,), jnp.float32),
            pltpu.VMEM((rows_pt,), jnp.float32),
            pltpu.VMEM_SHARED((NP,), jnp.float32),
        ],
    )
    def deg_kernel(dst_hbm, out_hbm, didx, ones_v, zeros_v, stage_v, acc_sp):
        c = lax.axis_index("c")
        s = lax.axis_index("s")
        w = c * _NS + s
        for j in range(_CHUNK // 16):
            ones_v[pl.ds(j * 16, 16)] = jnp.full((16,), 1.0, jnp.float32)
            zeros_v[pl.ds(j * 16, 16)] = jnp.zeros((16,), jnp.float32)
        base = pl.multiple_of(s * rows_pt, _CHUNK)
        for t in range(zc):
            pltpu.sync_copy(zeros_v, acc_sp.at[pl.ds(base + t * _CHUNK, _CHUNK)])
        plsc.subcore_barrier()
        pltpu.sync_copy(dst_hbm.at[pl.ds(w * CPW, CPW)], didx)

        def body(ci, carry):
            pltpu.sync_copy(ones_v, acc_sp.at[didx.at[ci]], add=True)
            return carry

        lax.fori_loop(0, CPW, body, 0)
        plsc.subcore_barrier()
        pltpu.sync_copy(acc_sp.at[pl.ds(base, rows_pt)], stage_v)
        pltpu.sync_copy(stage_v, out_hbm.at[c, pl.ds(base, rows_pt)])

    return deg_kernel


def _make_agg_kernel(NP, H, CPW):
    rows_pt = NP // _NS
    zc = rows_pt // _CHUNK

    NB = 4

    @functools.partial(
        pl.kernel,
        out_type=jax.ShapeDtypeStruct((_NC, NP, H), jnp.float32),
        mesh=_sc_mesh(),
        compiler_params=pltpu.CompilerParams(use_tc_tiling_on_sc=False),
        scratch_types=[
            pltpu.VMEM((CPW, _CHUNK), jnp.int32),
            pltpu.VMEM((CPW, _CHUNK), jnp.int32),
            [pltpu.VMEM((_CHUNK, H), jnp.float32) for _ in range(NB)],
            pltpu.VMEM_SHARED((NP, H), jnp.float32),
            [pltpu.SemaphoreType.DMA for _ in range(NB)],
            [pltpu.SemaphoreType.DMA for _ in range(NB)],
        ],
    )
    def agg_kernel(u_hbm, src_hbm, dst_hbm, out_hbm,
                   sidx, didx, bufs, acc_sp, gsems, tsems):
        c = lax.axis_index("c")
        s = lax.axis_index("s")
        w = c * _NS + s

        def zrow(i, carry):
            for j in range(H // 16):
                bufs[0][i, pl.ds(j * 16, 16)] = jnp.zeros((16,), jnp.float32)
            return carry

        lax.fori_loop(0, _CHUNK, zrow, 0)
        base = pl.multiple_of(s * rows_pt, _CHUNK)
        for t in range(zc):
            pltpu.sync_copy(bufs[0],
                            acc_sp.at[pl.ds(base + t * _CHUNK, _CHUNK)])
        plsc.subcore_barrier()
        pltpu.sync_copy(src_hbm.at[pl.ds(w * CPW, CPW)], sidx)
        pltpu.sync_copy(dst_hbm.at[pl.ds(w * CPW, CPW)], didx)

        rounds = CPW // NB

        def gathers(p):
            for j in range(NB):
                pltpu.async_copy(u_hbm.at[sidx.at[p * NB + j]], bufs[j],
                                 gsems[j])

        def scatters(p):
            for j in range(NB):
                pltpu.make_async_copy(u_hbm.at[sidx.at[0]], bufs[j],
                                      gsems[j]).wait()
                pltpu.async_copy(bufs[j], acc_sp.at[didx.at[p * NB + j]],
                                 tsems[j], add=True)

        gathers(0)

        def body(p, carry):
            scatters(p)
            for j in range(NB):
                pltpu.make_async_copy(bufs[j], acc_sp.at[didx.at[0]],
                                      tsems[j]).wait()
                pltpu.async_copy(u_hbm.at[sidx.at[(p + 1) * NB + j]],
                                 bufs[j], gsems[j])
            return carry

        lax.fori_loop(0, rounds - 1, body, 0)
        scatters(rounds - 1)
        for j in range(NB):
            pltpu.make_async_copy(bufs[j], acc_sp.at[didx.at[0]],
                                  tsems[j]).wait()
        plsc.subcore_barrier()

        def copyout(t, carry):
            off = pl.multiple_of(base + t * _CHUNK, _CHUNK)
            pltpu.sync_copy(acc_sp.at[pl.ds(off, _CHUNK)], bufs[0])
            pltpu.sync_copy(bufs[0], out_hbm.at[c, pl.ds(off, _CHUNK)])
            return carry

        lax.fori_loop(0, zc, copyout, 0)

    return agg_kernel


def _tc_pre(xp, W1, deg0, deg1, NP, N):
    D = xp.shape[1]
    H = W1.shape[1]

    def body(x_ref, w_ref, d0_ref, d1_ref, dinv_ref, u_ref):
        i = pl.program_id(0)
        d = d0_ref[...] + d1_ref[...] + 1.0
        rows = lax.broadcasted_iota(jnp.int32, (_R, 1), 0) + i * _R
        dinv = jnp.where(rows < N, lax.rsqrt(d), 0.0)
        dinv_ref[...] = dinv
        u_ref[...] = dinv * jnp.dot(x_ref[...], w_ref[...],
                                    preferred_element_type=jnp.float32)

    return pl.pallas_call(
        body,
        grid=(NP // _R,),
        in_specs=[
            pl.BlockSpec((_R, D), lambda i: (i, 0)),
            pl.BlockSpec((D, H), lambda i: (0, 0)),
            pl.BlockSpec((_R, 1), lambda i: (i, 0)),
            pl.BlockSpec((_R, 1), lambda i: (i, 0)),
        ],
        out_specs=[
            pl.BlockSpec((_R, 1), lambda i: (i, 0)),
            pl.BlockSpec((_R, H), lambda i: (i, 0)),
        ],
        out_shape=[
            jax.ShapeDtypeStruct((NP, 1), jnp.float32),
            jax.ShapeDtypeStruct((NP, H), jnp.float32),
        ],
    )(xp, W1, deg0, deg1)


def _tc_layer(p, u, dinv, b, Wn, NP):
    H = u.shape[1]
    Hn = Wn.shape[1]

    def body(p0_ref, p1_ref, u_ref, dinv_ref, b_ref, w_ref, un_ref):
        dinv = dinv_ref[...]
        h = jnp.maximum((p0_ref[...] + p1_ref[...] + u_ref[...]) * dinv
                        + b_ref[...], 0.0)
        un_ref[...] = dinv * jnp.dot(h, w_ref[...],
                                     preferred_element_type=jnp.float32)

    return pl.pallas_call(
        body,
        grid=(NP // _R,),
        in_specs=[
            pl.BlockSpec((_R, H), lambda i: (i, 0)),
            pl.BlockSpec((_R, H), lambda i: (i, 0)),
            pl.BlockSpec((_R, H), lambda i: (i, 0)),
            pl.BlockSpec((_R, 1), lambda i: (i, 0)),
            pl.BlockSpec((1, H), lambda i: (0, 0)),
            pl.BlockSpec((H, Hn), lambda i: (0, 0)),
        ],
        out_specs=pl.BlockSpec((_R, Hn), lambda i: (i, 0)),
        out_shape=jax.ShapeDtypeStruct((NP, Hn), jnp.float32),
    )(p[0], p[1], u, dinv, b, Wn)


def _tc_head(p, u, dinv, b3, Wp1, bp1, Wp2, bp2, NP):
    H = u.shape[1]
    Hh = Wp1.shape[1]

    def body(p0_ref, p1_ref, u_ref, dinv_ref, b3_ref, w1_ref, b1_ref,
             w2_ref, b2_ref, out_ref):
        dinv = dinv_ref[...]
        h3 = jnp.maximum((p0_ref[...] + p1_ref[...] + u_ref[...]) * dinv
                         + b3_ref[...], 0.0)
        hp = jnp.maximum(jnp.dot(h3, w1_ref[...],
                                 preferred_element_type=jnp.float32)
                         + b1_ref[...], 0.0)
        out_ref[...] = jnp.dot(hp, w2_ref[...],
                               preferred_element_type=jnp.float32) + b2_ref[...]

    return pl.pallas_call(
        body,
        grid=(NP // _R,),
        in_specs=[
            pl.BlockSpec((_R, H), lambda i: (i, 0)),
            pl.BlockSpec((_R, H), lambda i: (i, 0)),
            pl.BlockSpec((_R, H), lambda i: (i, 0)),
            pl.BlockSpec((_R, 1), lambda i: (i, 0)),
            pl.BlockSpec((1, H), lambda i: (0, 0)),
            pl.BlockSpec((H, Hh), lambda i: (0, 0)),
            pl.BlockSpec((1, Hh), lambda i: (0, 0)),
            pl.BlockSpec((Hh, 1), lambda i: (0, 0)),
            pl.BlockSpec((1, 1), lambda i: (0, 0)),
        ],
        out_specs=pl.BlockSpec((_R, 1), lambda i: (i, 0)),
        out_shape=jax.ShapeDtypeStruct((NP, 1), jnp.float32),
    )(p[0], p[1], u, dinv, b3, Wp1, bp1, Wp2, bp2)


def kernel(x, edge_index, batch, W1, b1, W2, b2, W3, b3, Wp1, bp1, Wp2, bp2):
    N, D = x.shape
    H = W1.shape[1]
    E = edge_index.shape[1]

    NP = ((N + 1 + _R - 1) // _R) * _R
    grain = _NW * _CHUNK * 8
    EP = ((E + grain - 1) // grain) * grain
    CPW = EP // (_NW * _CHUNK)

    xp = jnp.pad(x, ((0, NP - N), (0, 0)))
    pad = EP - E
    pad_ids = N + jnp.arange(pad, dtype=jnp.int32) % (NP - N)
    srcp = jnp.concatenate(
        [edge_index[0], pad_ids]).reshape(EP // _CHUNK, _CHUNK)
    dstp = jnp.concatenate(
        [edge_index[1], pad_ids]).reshape(EP // _CHUNK, _CHUNK)
    b1r = b1.reshape(1, H)
    b2r = b2.reshape(1, H)
    b3r = b3.reshape(1, H)
    bp1r = bp1.reshape(1, -1)
    bp2r = bp2.reshape(1, 1)

    deg_p = _make_deg_kernel(NP, CPW)(dstp)
    deg01 = deg_p.reshape(_NC, NP, 1)

    dinv, u1 = _tc_pre(xp, W1, deg01[0], deg01[1], NP, N)

    agg = _make_agg_kernel(NP, H, CPW)

    p1_ = agg(u1, srcp, dstp)
    u2 = _tc_layer(p1_, u1, dinv, b1r, W2, NP)
    p2_ = agg(u2, srcp, dstp)
    u3 = _tc_layer(p2_, u2, dinv, b2r, W3, NP)
    p3_ = agg(u3, srcp, dstp)
    out = _tc_head(p3_, u3, dinv, b3r, Wp1, bp1r, Wp2, bp2r, NP)
    return out[:N]
